# baseline (speedup 1.0000x reference)
import functools

import jax
import jax.numpy as jnp
from jax import lax
from jax.experimental import pallas as pl
from jax.experimental.pallas import tpu as pltpu
from jax.experimental.pallas import tpu_sc as plsc

_L = 16

_LN2 = 0.6931471805599453
_SQRT2 = 1.4142135623730951


def _ln(x):
    xi = plsc.bitcast(x, jnp.int32)
    e = (xi >> 23) - 127
    mi = (xi & 0x007FFFFF) | 0x3F800000
    m = plsc.bitcast(mi, jnp.float32)
    big = m > _SQRT2
    m = jnp.where(big, m * 0.5, m)
    e = jnp.where(big, e + 1, e)
    t = (m - 1.0) / (m + 1.0)
    t2 = t * t
    p = t * (2.0 + t2 * (2.0 / 3.0 + t2 * (2.0 / 5.0 + t2 * (2.0 / 7.0 + t2 * (2.0 / 9.0)))))
    return p + e.astype(jnp.float32) * _LN2


def _make_sc_kernel(B, M, M_pad, N_pad, apw, num_cores):
    nchunks = apw // _L
    nt = M_pad // _L
    TBS = 5 * M_pad

    mesh = plsc.VectorSubcoreMesh(core_axis_name="c", subcore_axis_name="s")

    @functools.partial(
        pl.kernel,
        mesh=mesh,
        compiler_params=pltpu.CompilerParams(needs_layout_passes=False),
        out_type=jax.ShapeDtypeStruct((B * N_pad * 5,), jnp.float32),
        scratch_types=[
            pltpu.VMEM((B * TBS,), jnp.float32),
            pltpu.VMEM((B * TBS,), jnp.float32),
            pltpu.VMEM((apw,), jnp.float32),
            pltpu.VMEM((apw,), jnp.float32),
            pltpu.VMEM((apw,), jnp.float32),
            pltpu.VMEM((apw,), jnp.float32),
            pltpu.VMEM((apw * 5,), jnp.float32),
        ],
    )
    def k(bt_hbm, x1_hbm, y1_hbm, x2_hbm, y2_hbm, out_hbm,
          bt_v, mbt_v, x1_v, y1_v, x2_v, y2_v, out_v):
        wid = lax.axis_index("s") * num_cores + lax.axis_index("c")
        base = wid * apw

        pltpu.sync_copy(bt_hbm, bt_v)
        pltpu.sync_copy(x1_hbm.at[pl.ds(base, apw)], x1_v)
        pltpu.sync_copy(y1_hbm.at[pl.ds(base, apw)], y1_v)
        pltpu.sync_copy(x2_hbm.at[pl.ds(base, apw)], x2_v)
        pltpu.sync_copy(y2_hbm.at[pl.ds(base, apw)], y2_v)

        for b in range(B):
            for t in range(nt):
                cls = bt_v[pl.ds(b * TBS + 4 * M_pad + t * _L, _L)]
                msk = cls != 0.0
                for c in range(5):
                    i0 = b * TBS + c * M_pad + t * _L
                    mbt_v[pl.ds(i0, _L)] = jnp.where(msk, bt_v[pl.ds(i0, _L)], 0.0)

        iota = lax.iota(jnp.int32, _L)
        iota5 = iota * 5

        W = 4
        nsc = nchunks // W

        for b in range(B):
            tb = b * TBS

            def sck_body(sc, _, tb=tb):
                o64 = pl.multiple_of(sc * (W * _L), W * _L)
                dx1 = [x1_v[pl.ds(o64 + j * _L, _L)] for j in range(W)]
                dy1 = [y1_v[pl.ds(o64 + j * _L, _L)] for j in range(W)]
                dx2 = [x2_v[pl.ds(o64 + j * _L, _L)] for j in range(W)]
                dy2 = [y2_v[pl.ds(o64 + j * _L, _L)] for j in range(W)]
                a2 = [dx2[j] * dy2[j] for j in range(W)]

                iab0 = tuple(jnp.full((_L,), -1.0, jnp.float32) for _ in range(W))
                ub0 = tuple(jnp.full((_L,), 1.0, jnp.float32) for _ in range(W))
                bidx0 = tuple(jnp.zeros((_L,), jnp.int32) for _ in range(W))

                @plsc.parallel_loop(0, M, unroll=2, carry=(iab0, ub0, bidx0))
                def m_loop(m, carry):
                    iabs, ubs, bidxs = carry
                    mb = jnp.full((_L,), m, jnp.int32)
                    tx1 = plsc.load_gather(mbt_v, [mb + (tb + 0 * M_pad)])
                    ty1 = plsc.load_gather(mbt_v, [mb + (tb + 1 * M_pad)])
                    tx2 = plsc.load_gather(mbt_v, [mb + (tb + 2 * M_pad)])
                    ty2 = plsc.load_gather(mbt_v, [mb + (tb + 3 * M_pad)])
                    a1 = tx2 * ty2
                    niabs, nubs, nbidxs = [], [], []
                    for j in range(W):
                        iwx = jnp.maximum(
                            jnp.minimum(tx2, dx2[j]) - jnp.maximum(tx1, dx1[j]), 0.0)
                        iwy = jnp.maximum(
                            jnp.minimum(ty2, dy2[j]) - jnp.maximum(ty1, dy1[j]), 0.0)
                        ia = iwx * iwy
                        u = (a1 + a2[j]) - ia
                        pred = ia * ubs[j] > iabs[j] * u
                        niabs.append(jnp.where(pred, ia, iabs[j]))
                        nubs.append(jnp.where(pred, u, ubs[j]))
                        nbidxs.append(jnp.where(pred, mb, bidxs[j]))
                    return tuple(niabs), tuple(nubs), tuple(nbidxs)

                iabs, ubs, bidxs = m_loop

                for j in range(W):
                    cur, bidx = iabs[j] / ubs[j], bidxs[j]
                    gx1 = plsc.load_gather(mbt_v, [bidx + (tb + 0 * M_pad)])
                    gy1 = plsc.load_gather(mbt_v, [bidx + (tb + 1 * M_pad)])
                    gx2 = plsc.load_gather(mbt_v, [bidx + (tb + 2 * M_pad)])
                    gy2 = plsc.load_gather(mbt_v, [bidx + (tb + 3 * M_pad)])
                    gcl = plsc.load_gather(mbt_v, [bidx + (tb + 4 * M_pad)])

                    pw = dx2[j] - dx1[j]
                    ph = dy2[j] - dy1[j]
                    o0 = ((gx2 + gx1) * 0.5 - (dx2[j] + dx1[j]) * 0.5) / pw
                    o1 = ((gy2 + gy1) * 0.5 - (dy2[j] + dy1[j]) * 0.5) / ph
                    o2 = _ln((gx2 - gx1) / pw)
                    o3 = _ln((gy2 - gy1) / ph)
                    o4 = gcl * jnp.where(cur >= 0.5, 1.0, 0.0)

                    ib = iota5 + (o64 + j * _L) * 5
                    plsc.store_scatter(out_v, [ib], o0)
                    plsc.store_scatter(out_v, [ib + 1], o1)
                    plsc.store_scatter(out_v, [ib + 2], o2)
                    plsc.store_scatter(out_v, [ib + 3], o3)
                    plsc.store_scatter(out_v, [ib + 4], o4)
                return 0

            lax.fori_loop(0, nsc, sck_body, 0)
            pltpu.sync_copy(out_v, out_hbm.at[pl.ds(b * N_pad * 5 + base * 5, apw * 5)])

    return k


def kernel(labels, default_boxes):
    B, M, C = labels.shape
    N = default_boxes.shape[0]
    info = plsc.get_sparse_core_info()
    num_cores, num_subcores = info.num_cores, info.num_subcores
    NW = num_cores * num_subcores

    apw = -(-N // (NW * 4 * _L)) * 4 * _L
    N_pad = apw * NW
    M_pad = -(-M // _L) * _L

    pad_row = jnp.array([[0.0, 0.0, 1.0, 1.0]], jnp.float32)
    db = jnp.concatenate(
        [default_boxes, jnp.tile(pad_row, (N_pad - N, 1))], axis=0)
    x1, y1, x2, y2 = (db[:, i] for i in range(4))

    bt = jnp.transpose(labels, (0, 2, 1))
    bt = jnp.pad(bt, ((0, 0), (0, 0), (0, M_pad - M)))
    bt_flat = jnp.reshape(bt, (-1,))

    k = _make_sc_kernel(B, M, M_pad, N_pad, apw, num_cores)
    out_flat = k(bt_flat, x1, y1, x2, y2)
    out = jnp.reshape(out_flat, (B, N_pad, 5))
    return out[:, :N, :]


# speedup vs baseline: 1.4402x; 1.4402x over previous
import functools

import jax
import jax.numpy as jnp
from jax import lax
from jax.experimental import pallas as pl
from jax.experimental.pallas import tpu as pltpu
from jax.experimental.pallas import tpu_sc as plsc

_L = 16

_LN2 = 0.6931471805599453
_SQRT2 = 1.4142135623730951


def _ln(x):
    xi = plsc.bitcast(x, jnp.int32)
    e = (xi >> 23) - 127
    mi = (xi & 0x007FFFFF) | 0x3F800000
    m = plsc.bitcast(mi, jnp.float32)
    big = m > _SQRT2
    m = jnp.where(big, m * 0.5, m)
    e = jnp.where(big, e + 1, e)
    t = (m - 1.0) / (m + 1.0)
    t2 = t * t
    p = t * (2.0 + t2 * (2.0 / 3.0 + t2 * (2.0 / 5.0 + t2 * (2.0 / 7.0 + t2 * (2.0 / 9.0)))))
    return p + e.astype(jnp.float32) * _LN2


def _make_sc_kernel(B, M, M_pad, N, apw, num_cores):
    nchunks = apw // _L
    nt = M_pad // _L
    TBS = 5 * M_pad

    mesh = plsc.VectorSubcoreMesh(core_axis_name="c", subcore_axis_name="s")

    @functools.partial(
        pl.kernel,
        mesh=mesh,
        compiler_params=pltpu.CompilerParams(needs_layout_passes=False),
        out_type=jax.ShapeDtypeStruct((B, N, 5), jnp.float32),
        scratch_types=[
            pltpu.VMEM((B * TBS,), jnp.float32),
            pltpu.VMEM((B * TBS,), jnp.float32),
            pltpu.VMEM((apw,), jnp.float32),
            pltpu.VMEM((apw,), jnp.float32),
            pltpu.VMEM((apw,), jnp.float32),
            pltpu.VMEM((apw,), jnp.float32),
            pltpu.VMEM((apw, 5), jnp.float32),
        ],
    )
    def k(bt_hbm, x1_hbm, y1_hbm, x2_hbm, y2_hbm, out_hbm,
          bt_v, mbt_v, x1_v, y1_v, x2_v, y2_v, out_v):
        wid = lax.axis_index("s") * num_cores + lax.axis_index("c")
        base = pl.multiple_of(jnp.minimum(wid * apw, N - apw), 8)

        pltpu.sync_copy(bt_hbm, bt_v)
        pltpu.sync_copy(x1_hbm.at[pl.ds(base, apw)], x1_v)
        pltpu.sync_copy(y1_hbm.at[pl.ds(base, apw)], y1_v)
        pltpu.sync_copy(x2_hbm.at[pl.ds(base, apw)], x2_v)
        pltpu.sync_copy(y2_hbm.at[pl.ds(base, apw)], y2_v)

        for b in range(B):
            for t in range(nt):
                cls = bt_v[pl.ds(b * TBS + 4 * M_pad + t * _L, _L)]
                msk = cls != 0.0
                for c in range(5):
                    i0 = b * TBS + c * M_pad + t * _L
                    mbt_v[pl.ds(i0, _L)] = jnp.where(msk, bt_v[pl.ds(i0, _L)], 0.0)

        iota = lax.iota(jnp.int32, _L)

        W = 4
        nsc = nchunks // W

        for b in range(B):
            tb = b * TBS

            def sck_body(sc, _, tb=tb):
                o64 = pl.multiple_of(sc * (W * _L), W * _L)
                dx1 = [x1_v[pl.ds(o64 + j * _L, _L)] for j in range(W)]
                dy1 = [y1_v[pl.ds(o64 + j * _L, _L)] for j in range(W)]
                dx2 = [x2_v[pl.ds(o64 + j * _L, _L)] for j in range(W)]
                dy2 = [y2_v[pl.ds(o64 + j * _L, _L)] for j in range(W)]
                a2 = [dx2[j] * dy2[j] for j in range(W)]

                iab0 = tuple(jnp.full((_L,), -1.0, jnp.float32) for _ in range(W))
                ub0 = tuple(jnp.full((_L,), 1.0, jnp.float32) for _ in range(W))
                bidx0 = tuple(jnp.zeros((_L,), jnp.int32) for _ in range(W))

                @plsc.parallel_loop(0, M, unroll=2, carry=(iab0, ub0, bidx0))
                def m_loop(m, carry):
                    iabs, ubs, bidxs = carry
                    mb = jnp.full((_L,), m, jnp.int32)
                    tx1 = plsc.load_gather(mbt_v, [mb + (tb + 0 * M_pad)])
                    ty1 = plsc.load_gather(mbt_v, [mb + (tb + 1 * M_pad)])
                    tx2 = plsc.load_gather(mbt_v, [mb + (tb + 2 * M_pad)])
                    ty2 = plsc.load_gather(mbt_v, [mb + (tb + 3 * M_pad)])
                    a1 = tx2 * ty2
                    niabs, nubs, nbidxs = [], [], []
                    for j in range(W):
                        iwx = jnp.maximum(
                            jnp.minimum(tx2, dx2[j]) - jnp.maximum(tx1, dx1[j]), 0.0)
                        iwy = jnp.maximum(
                            jnp.minimum(ty2, dy2[j]) - jnp.maximum(ty1, dy1[j]), 0.0)
                        ia = iwx * iwy
                        u = (a1 + a2[j]) - ia
                        pred = ia * ubs[j] > iabs[j] * u
                        niabs.append(jnp.where(pred, ia, iabs[j]))
                        nubs.append(jnp.where(pred, u, ubs[j]))
                        nbidxs.append(jnp.where(pred, mb, bidxs[j]))
                    return tuple(niabs), tuple(nubs), tuple(nbidxs)

                iabs, ubs, bidxs = m_loop

                for j in range(W):
                    cur, bidx = iabs[j] / ubs[j], bidxs[j]
                    gx1 = plsc.load_gather(mbt_v, [bidx + (tb + 0 * M_pad)])
                    gy1 = plsc.load_gather(mbt_v, [bidx + (tb + 1 * M_pad)])
                    gx2 = plsc.load_gather(mbt_v, [bidx + (tb + 2 * M_pad)])
                    gy2 = plsc.load_gather(mbt_v, [bidx + (tb + 3 * M_pad)])
                    gcl = plsc.load_gather(mbt_v, [bidx + (tb + 4 * M_pad)])

                    pw = dx2[j] - dx1[j]
                    ph = dy2[j] - dy1[j]
                    o0 = ((gx2 + gx1) * 0.5 - (dx2[j] + dx1[j]) * 0.5) / pw
                    o1 = ((gy2 + gy1) * 0.5 - (dy2[j] + dy1[j]) * 0.5) / ph
                    o2 = _ln((gx2 - gx1) / pw)
                    o3 = _ln((gy2 - gy1) / ph)
                    o4 = gcl * jnp.where(cur >= 0.5, 1.0, 0.0)

                    rows = iota + (o64 + j * _L)
                    for c, o in enumerate((o0, o1, o2, o3, o4)):
                        plsc.store_scatter(out_v, [rows, jnp.full((_L,), c, jnp.int32)], o)
                return 0

            lax.fori_loop(0, nsc, sck_body, 0)
            pltpu.sync_copy(out_v, out_hbm.at[b, pl.ds(base, apw)])

    return k


def kernel(labels, default_boxes):
    B, M, C = labels.shape
    N = default_boxes.shape[0]
    info = plsc.get_sparse_core_info()
    num_cores, num_subcores = info.num_cores, info.num_subcores
    NW = num_cores * num_subcores

    apw = -(-N // (NW * 4 * _L)) * 4 * _L
    M_pad = -(-M // _L) * _L
    assert N % 8 == 0 and N >= apw

    x1, y1, x2, y2 = (default_boxes[:, i] for i in range(4))

    bt = jnp.transpose(labels, (0, 2, 1))
    bt = jnp.pad(bt, ((0, 0), (0, 0), (0, M_pad - M)))
    bt_flat = jnp.reshape(bt, (-1,))

    k = _make_sc_kernel(B, M, M_pad, N, apw, num_cores)
    return k(bt_flat, x1, y1, x2, y2)
